# Initial kernel scaffold; baseline (speedup 1.0000x reference)
import jax
import jax.numpy as jnp
from jax.experimental import pallas as pl


def kernel(features_plus, features_minus, edge_index_pos, edge_index_neg, Wp1, bp1, Wp2, bp2, Wn1, bn1, Wn2, bn2):
    raise NotImplementedError("write your pallas kernel here")


# speedup vs baseline: 14.3107x; 14.3107x over previous
import functools

import jax
import jax.numpy as jnp
from jax import lax
from jax.experimental import pallas as pl
from jax.experimental.pallas import tpu as pltpu
from jax.experimental.pallas import tpu_sc as plsc

N = 10000
D = 128
H = 64
E = 320000

NC = 2
NS = 16
NW = NC * NS
CH = 128
NCHUNK = E // CH
TPG = -(-NCHUNK // NW)
RPS = N // NS
ZCH = 1000
ZR = 200
BLK = 1000

_mesh = plsc.VectorSubcoreMesh(core_axis_name="c", subcore_axis_name="s")
_sc_params = pltpu.CompilerParams(use_tc_tiling_on_sc=False)


@functools.partial(
    pl.kernel,
    out_type=jax.ShapeDtypeStruct((NC, 2, 1, N), jnp.float32),
    mesh=_mesh,
    compiler_params=_sc_params,
    scratch_types=[
        pltpu.VMEM((CH,), jnp.int32),
        pltpu.VMEM((CH,), jnp.float32),
        pltpu.VMEM((ZCH,), jnp.float32),
        pltpu.VMEM((N,), jnp.float32),
        pltpu.VMEM_SHARED((N,), jnp.float32),
        pltpu.VMEM_SHARED((N,), jnp.float32),
        pltpu.SemaphoreType.DMA,
    ],
)
def _deg_kernel(dstp_hbm, dstn_hbm, ones_hbm, zeros_hbm, out_hbm,
                idx_v, ones_v, zb, wb, accp, accn, sem):
    c = lax.axis_index("c")
    s = lax.axis_index("s")
    wid = s * NC + c

    pltpu.sync_copy(zeros_hbm, zb)
    for g in range(2):
        acc = accp if g == 0 else accn
        for j in range(N // ZCH):
            owner = (g * (N // ZCH) + j) % NS

            @pl.when(s == owner)
            def _(acc=acc, j=j):
                pltpu.sync_copy(zb, acc.at[pl.ds(j * ZCH, ZCH)])

    pltpu.sync_copy(ones_hbm, ones_v)
    plsc.subcore_barrier()

    def body(t, carry):
        cid = wid + NW * t

        @pl.when(cid < NCHUNK)
        def _():
            off = pl.multiple_of(cid * CH, CH)
            pltpu.sync_copy(dstp_hbm.at[pl.ds(off, CH)], idx_v)
            pltpu.sync_copy(ones_v, accp.at[idx_v], add=True)
            pltpu.sync_copy(dstn_hbm.at[pl.ds(off, CH)], idx_v)
            pltpu.sync_copy(ones_v, accn.at[idx_v], add=True)

        return carry

    lax.fori_loop(0, TPG, body, 0)
    plsc.subcore_barrier()

    for g in range(2):
        acc = accp if g == 0 else accn

        @pl.when(s == g)
        def _(acc=acc, g=g):
            pltpu.sync_copy(acc, wb)
            pltpu.sync_copy(wb, out_hbm.at[c, g, 0])


@functools.partial(
    pl.kernel,
    out_type=jax.ShapeDtypeStruct((2, NC, N, H), jnp.float32),
    mesh=_mesh,
    compiler_params=_sc_params,
    scratch_types=[
        pltpu.VMEM((CH,), jnp.int32),
        pltpu.VMEM((CH,), jnp.int32),
        pltpu.VMEM((CH, H), jnp.float32),
        pltpu.VMEM((ZR, H), jnp.float32),
        pltpu.VMEM_SHARED((N, H), jnp.float32),
        pltpu.VMEM_SHARED((N, H), jnp.float32),
        pltpu.SemaphoreType.DMA,
    ],
)
def _scatter_kernel(tabp_hbm, tabn_hbm, srcp_hbm, dstp_hbm, srcn_hbm,
                    dstn_hbm, zrows_hbm, out_hbm,
                    idx_s, idx_d, rows, zb, accp, accn, sem):
    c = lax.axis_index("c")
    s = lax.axis_index("s")
    wid = s * NC + c

    pltpu.sync_copy(zrows_hbm, zb)
    for g in range(2):
        acc = accp if g == 0 else accn
        for j in range(N // ZR):
            owner = (g * (N // ZR) + j) % NS

            @pl.when(s == owner)
            def _(acc=acc, j=j):
                pltpu.sync_copy(zb, acc.at[pl.ds(j * ZR, ZR)])

    plsc.subcore_barrier()

    for g in range(2):
        tab = tabp_hbm if g == 0 else tabn_hbm
        src = srcp_hbm if g == 0 else srcn_hbm
        dst = dstp_hbm if g == 0 else dstn_hbm
        acc = accp if g == 0 else accn

        def body(t, carry, tab=tab, src=src, dst=dst, acc=acc):
            cid = wid + NW * t

            @pl.when(cid < NCHUNK)
            def _():
                off = pl.multiple_of(cid * CH, CH)
                pltpu.sync_copy(src.at[pl.ds(off, CH)], idx_s)
                pltpu.sync_copy(dst.at[pl.ds(off, CH)], idx_d)
                pltpu.async_copy(tab.at[idx_s], rows, sem).wait()
                pltpu.sync_copy(rows, acc.at[idx_d], add=True)

            return carry

        lax.fori_loop(0, TPG, body, 0)

    plsc.subcore_barrier()
    for g in range(2):
        acc = accp if g == 0 else accn
        for j in range(N // ZR):
            owner = (g * (N // ZR) + j) % NS

            @pl.when(s == owner)
            def _(acc=acc, g=g, j=j):
                pltpu.sync_copy(acc.at[pl.ds(j * ZR, ZR)], zb)
                pltpu.sync_copy(zb, out_hbm.at[g, c, pl.ds(j * ZR, ZR)])


def _dinv(dp):
    deg = dp[:, 0:1] + dp[:, 1:2] + 1.0
    return lax.rsqrt(deg)


def _tc_a_body(x_ref, w_ref, b_ref, dp_ref, h_ref, g_ref):
    h = jnp.dot(x_ref[...], w_ref[...],
                preferred_element_type=jnp.float32) + b_ref[...]
    dinv = _dinv(dp_ref[...])
    h_ref[...] = h
    g_ref[...] = dinv * h


_tc_a = pl.pallas_call(
    _tc_a_body,
    grid=(N // BLK,),
    in_specs=[
        pl.BlockSpec((BLK, D), lambda i: (i, 0)),
        pl.BlockSpec((D, H), lambda i: (0, 0)),
        pl.BlockSpec((1, H), lambda i: (0, 0)),
        pl.BlockSpec((BLK, 2), lambda i: (i, 0)),
    ],
    out_specs=[pl.BlockSpec((BLK, H), lambda i: (i, 0))] * 2,
    out_shape=[jax.ShapeDtypeStruct((N, H), jnp.float32)] * 2,
)


def _tc_b_body(sp_ref, h1_ref, dp_ref, w_ref, b_ref, h2_ref, g2_ref):
    dinv = _dinv(dp_ref[...])
    ssum = sp_ref[0] + sp_ref[1]
    z = jnp.maximum(dinv * ssum + (dinv * dinv) * h1_ref[...], 0.0)
    h2 = jnp.dot(z, w_ref[...],
                 preferred_element_type=jnp.float32) + b_ref[...]
    h2_ref[...] = h2
    g2_ref[...] = dinv * h2


_tc_b = pl.pallas_call(
    _tc_b_body,
    grid=(N // BLK,),
    in_specs=[
        pl.BlockSpec((NC, BLK, H), lambda i: (0, i, 0)),
        pl.BlockSpec((BLK, H), lambda i: (i, 0)),
        pl.BlockSpec((BLK, 2), lambda i: (i, 0)),
        pl.BlockSpec((H, H), lambda i: (0, 0)),
        pl.BlockSpec((1, H), lambda i: (0, 0)),
    ],
    out_specs=[pl.BlockSpec((BLK, H), lambda i: (i, 0))] * 2,
    out_shape=[jax.ShapeDtypeStruct((N, H), jnp.float32)] * 2,
)


def _tc_c_body(sp_ref, h2_ref, dp_ref, o_ref):
    dinv = _dinv(dp_ref[...])
    ssum = sp_ref[0] + sp_ref[1]
    o_ref[...] = dinv * ssum + (dinv * dinv) * h2_ref[...]


_tc_c = pl.pallas_call(
    _tc_c_body,
    grid=(N // BLK,),
    in_specs=[
        pl.BlockSpec((NC, BLK, H), lambda i: (0, i, 0)),
        pl.BlockSpec((BLK, H), lambda i: (i, 0)),
        pl.BlockSpec((BLK, 2), lambda i: (i, 0)),
    ],
    out_specs=pl.BlockSpec((BLK, H), lambda i: (i, 0)),
    out_shape=jax.ShapeDtypeStruct((N, H), jnp.float32),
)


def kernel(features_plus, features_minus, edge_index_pos, edge_index_neg,
           Wp1, bp1, Wp2, bp2, Wn1, bn1, Wn2, bn2):
    srcp = edge_index_pos[0]
    dstp = edge_index_pos[1]
    srcn = edge_index_neg[0]
    dstn = edge_index_neg[1]
    ones = jnp.ones((CH,), jnp.float32)
    zeros1 = jnp.zeros((ZCH,), jnp.float32)
    zrows = jnp.zeros((ZR, H), jnp.float32)

    degp = _deg_kernel(dstp, dstn, ones, zeros1)
    dp = jnp.transpose(degp[:, :, 0, :], (1, 2, 0))

    h1p, g1p = _tc_a(features_plus, Wp1, bp1.reshape(1, H), dp[0])
    h1n, g1n = _tc_a(features_minus, Wn1, bn1.reshape(1, H), dp[1])

    s1 = _scatter_kernel(g1p, g1n, srcp, dstp, srcn, dstn, zrows)

    h2p, g2p = _tc_b(s1[0], h1p, dp[0], Wp2, bp2.reshape(1, H))
    h2n, g2n = _tc_b(s1[1], h1n, dp[1], Wn2, bn2.reshape(1, H))

    s2 = _scatter_kernel(g2p, g2n, srcp, dstp, srcn, dstn, zrows)

    x = _tc_c(s2[0], h2p, dp[0])
    y = _tc_c(s2[1], h2n, dp[1])
    return (x, y)


# speedup vs baseline: 29.4102x; 2.0551x over previous
import functools

import jax
import jax.numpy as jnp
from jax import lax
from jax.experimental import pallas as pl
from jax.experimental.pallas import tpu as pltpu
from jax.experimental.pallas import tpu_sc as plsc

N = 10000
D = 128
H = 64
E = 320000

NC = 2
NS = 16
NW = NC * NS
GC = 125
NCH = E // GC // NW
DC = 1000
NDC = E // DC // NW
ZCH = 1000
ZR = GC
BLK = 1000

_mesh = plsc.VectorSubcoreMesh(core_axis_name="c", subcore_axis_name="s")
_sc_params = pltpu.CompilerParams(use_tc_tiling_on_sc=False)


@functools.partial(
    pl.kernel,
    out_type=jax.ShapeDtypeStruct((NC, 2, 1, N), jnp.float32),
    mesh=_mesh,
    compiler_params=_sc_params,
    scratch_types=[
        pltpu.VMEM((NDC, DC), jnp.int32),
        pltpu.VMEM((DC,), jnp.float32),
        pltpu.VMEM((ZCH,), jnp.float32),
        pltpu.VMEM((N,), jnp.float32),
        pltpu.VMEM_SHARED((N,), jnp.float32),
        pltpu.VMEM_SHARED((N,), jnp.float32),
        pltpu.SemaphoreType.DMA,
    ],
)
def _deg_kernel(dstp_hbm, dstn_hbm, ones_hbm, zeros_hbm, out_hbm,
                idx_v, ones_v, zb, wb, accp, accn, sem):
    c = lax.axis_index("c")
    s = lax.axis_index("s")
    wid = s * NC + c

    pltpu.sync_copy(zeros_hbm, zb)
    for g in range(2):
        acc = accp if g == 0 else accn
        for j in range(N // ZCH):
            owner = (g * (N // ZCH) + j) % NS

            @pl.when(s == owner)
            def _(acc=acc, j=j):
                pltpu.sync_copy(zb, acc.at[pl.ds(j * ZCH, ZCH)])

    pltpu.sync_copy(ones_hbm, ones_v)
    plsc.subcore_barrier()

    for g in range(2):
        dst = dstp_hbm if g == 0 else dstn_hbm
        acc = accp if g == 0 else accn
        pltpu.sync_copy(dst.at[pl.ds(wid * NDC, NDC)], idx_v)

        def body(j, carry, acc=acc):
            pltpu.sync_copy(ones_v, acc.at[idx_v.at[j]], add=True)
            return carry

        lax.fori_loop(0, NDC, body, 0)

    plsc.subcore_barrier()

    for g in range(2):
        acc = accp if g == 0 else accn

        @pl.when(s == g)
        def _(acc=acc, g=g):
            pltpu.sync_copy(acc, wb)
            pltpu.sync_copy(wb, out_hbm.at[c, g, 0])


@functools.partial(
    pl.kernel,
    out_type=jax.ShapeDtypeStruct((2, NC, N, H), jnp.float32),
    mesh=_mesh,
    compiler_params=_sc_params,
    scratch_types=[
        pltpu.VMEM((NCH, GC), jnp.int32),
        pltpu.VMEM((NCH, GC), jnp.int32),
        pltpu.VMEM((GC, H), jnp.float32),
        pltpu.VMEM((GC, H), jnp.float32),
        pltpu.VMEM_SHARED((N, H), jnp.float32),
        pltpu.VMEM_SHARED((N, H), jnp.float32),
        pltpu.SemaphoreType.DMA,
        pltpu.SemaphoreType.DMA,
        pltpu.SemaphoreType.DMA,
        pltpu.SemaphoreType.DMA,
    ],
)
def _scatter_kernel(tabp_hbm, tabn_hbm, srcp_hbm, dstp_hbm, srcn_hbm,
                    dstn_hbm, zrows_hbm, out_hbm,
                    idx_s, idx_d, rows0, rows1, accp, accn,
                    gsem0, gsem1, ssem0, ssem1):
    c = lax.axis_index("c")
    s = lax.axis_index("s")
    wid = s * NC + c
    rows = (rows0, rows1)
    gsem = (gsem0, gsem1)
    ssem = (ssem0, ssem1)

    pltpu.sync_copy(zrows_hbm, rows0)
    for g in range(2):
        acc = accp if g == 0 else accn
        for j in range(N // ZR):
            owner = (g * (N // ZR) + j) % NS

            @pl.when(s == owner)
            def _(acc=acc, j=j):
                pltpu.sync_copy(rows0, acc.at[pl.ds(j * ZR, ZR)])

    plsc.subcore_barrier()

    for g in range(2):
        tab = tabp_hbm if g == 0 else tabn_hbm
        src = srcp_hbm if g == 0 else srcn_hbm
        dst = dstp_hbm if g == 0 else dstn_hbm
        acc = accp if g == 0 else accn

        pltpu.sync_copy(src.at[pl.ds(wid * NCH, NCH)], idx_s)
        pltpu.sync_copy(dst.at[pl.ds(wid * NCH, NCH)], idx_d)

        pltpu.async_copy(tab.at[idx_s.at[0]], rows[0], gsem[0])

        def pair(p, carry, tab=tab, acc=acc):
            for b in range(2):
                t = 2 * p + b
                o = 1 - b

                @pl.when(t > 0)
                def _(b=b, o=o):
                    pltpu.make_async_copy(zrows_hbm, rows[o], ssem[o]).wait()

                @pl.when(t + 1 < NCH)
                def _(t=t, b=b, o=o, tab=tab):
                    pltpu.async_copy(tab.at[idx_s.at[t + 1]], rows[o],
                                     gsem[o])

                pltpu.make_async_copy(zrows_hbm, rows[b], gsem[b]).wait()
                pltpu.async_copy(rows[b], acc.at[idx_d.at[t]], ssem[b],
                                 add=True)
            return carry

        lax.fori_loop(0, NCH // 2, pair, 0)
        pltpu.make_async_copy(zrows_hbm, rows[1], ssem[1]).wait()

    plsc.subcore_barrier()
    for g in range(2):
        acc = accp if g == 0 else accn
        for j in range(N // ZR):
            owner = (g * (N // ZR) + j) % NS

            @pl.when(s == owner)
            def _(acc=acc, g=g, j=j):
                pltpu.sync_copy(acc.at[pl.ds(j * ZR, ZR)], rows0)
                pltpu.sync_copy(rows0, out_hbm.at[g, c, pl.ds(j * ZR, ZR)])


def _dinv(dp):
    deg = dp[:, 0:1] + dp[:, 1:2] + 1.0
    return lax.rsqrt(deg)


def _tc_a_body(x_ref, w_ref, b_ref, dp_ref, h_ref, g_ref):
    h = jnp.dot(x_ref[...], w_ref[...],
                preferred_element_type=jnp.float32) + b_ref[...]
    dinv = _dinv(dp_ref[...])
    h_ref[...] = h
    g_ref[...] = dinv * h


_tc_a = pl.pallas_call(
    _tc_a_body,
    grid=(N // BLK,),
    in_specs=[
        pl.BlockSpec((BLK, D), lambda i: (i, 0)),
        pl.BlockSpec((D, H), lambda i: (0, 0)),
        pl.BlockSpec((1, H), lambda i: (0, 0)),
        pl.BlockSpec((BLK, 2), lambda i: (i, 0)),
    ],
    out_specs=[pl.BlockSpec((BLK, H), lambda i: (i, 0))] * 2,
    out_shape=[jax.ShapeDtypeStruct((N, H), jnp.float32)] * 2,
)


def _tc_b_body(sp_ref, h1_ref, dp_ref, w_ref, b_ref, h2_ref, g2_ref):
    dinv = _dinv(dp_ref[...])
    ssum = sp_ref[0] + sp_ref[1]
    z = jnp.maximum(dinv * ssum + (dinv * dinv) * h1_ref[...], 0.0)
    h2 = jnp.dot(z, w_ref[...],
                 preferred_element_type=jnp.float32) + b_ref[...]
    h2_ref[...] = h2
    g2_ref[...] = dinv * h2


_tc_b = pl.pallas_call(
    _tc_b_body,
    grid=(N // BLK,),
    in_specs=[
        pl.BlockSpec((NC, BLK, H), lambda i: (0, i, 0)),
        pl.BlockSpec((BLK, H), lambda i: (i, 0)),
        pl.BlockSpec((BLK, 2), lambda i: (i, 0)),
        pl.BlockSpec((H, H), lambda i: (0, 0)),
        pl.BlockSpec((1, H), lambda i: (0, 0)),
    ],
    out_specs=[pl.BlockSpec((BLK, H), lambda i: (i, 0))] * 2,
    out_shape=[jax.ShapeDtypeStruct((N, H), jnp.float32)] * 2,
)


def _tc_c_body(sp_ref, h2_ref, dp_ref, o_ref):
    dinv = _dinv(dp_ref[...])
    ssum = sp_ref[0] + sp_ref[1]
    o_ref[...] = dinv * ssum + (dinv * dinv) * h2_ref[...]


_tc_c = pl.pallas_call(
    _tc_c_body,
    grid=(N // BLK,),
    in_specs=[
        pl.BlockSpec((NC, BLK, H), lambda i: (0, i, 0)),
        pl.BlockSpec((BLK, H), lambda i: (i, 0)),
        pl.BlockSpec((BLK, 2), lambda i: (i, 0)),
    ],
    out_specs=pl.BlockSpec((BLK, H), lambda i: (i, 0)),
    out_shape=jax.ShapeDtypeStruct((N, H), jnp.float32),
)


def kernel(features_plus, features_minus, edge_index_pos, edge_index_neg,
           Wp1, bp1, Wp2, bp2, Wn1, bn1, Wn2, bn2):
    srcp = edge_index_pos[0].reshape(E // GC, GC)
    dstp = edge_index_pos[1].reshape(E // GC, GC)
    srcn = edge_index_neg[0].reshape(E // GC, GC)
    dstn = edge_index_neg[1].reshape(E // GC, GC)
    dstp_d = edge_index_pos[1].reshape(E // DC, DC)
    dstn_d = edge_index_neg[1].reshape(E // DC, DC)
    ones = jnp.ones((DC,), jnp.float32)
    zeros1 = jnp.zeros((ZCH,), jnp.float32)
    zrows = jnp.zeros((ZR, H), jnp.float32)

    degp = _deg_kernel(dstp_d, dstn_d, ones, zeros1)
    dp = jnp.transpose(degp[:, :, 0, :], (1, 2, 0))

    h1p, g1p = _tc_a(features_plus, Wp1, bp1.reshape(1, H), dp[0])
    h1n, g1n = _tc_a(features_minus, Wn1, bn1.reshape(1, H), dp[1])

    s1 = _scatter_kernel(g1p, g1n, srcp, dstp, srcn, dstn, zrows)

    h2p, g2p = _tc_b(s1[0], h1p, dp[0], Wp2, bp2.reshape(1, H))
    h2n, g2n = _tc_b(s1[1], h1n, dp[1], Wn2, bn2.reshape(1, H))

    s2 = _scatter_kernel(g2p, g2n, srcp, dstp, srcn, dstn, zrows)

    x = _tc_c(s2[0], h2p, dp[0])
    y = _tc_c(s2[1], h2n, dp[1])
    return (x, y)


# speedup vs baseline: 37.6170x; 1.2790x over previous
import functools

import jax
import jax.numpy as jnp
from jax import lax
from jax.experimental import pallas as pl
from jax.experimental.pallas import tpu as pltpu
from jax.experimental.pallas import tpu_sc as plsc

N = 10000
D = 128
H = 64
E = 320000

NC = 2
NS = 16
NW = NC * NS
GC = 125
NCHG = E // GC
NCH = NCHG // NW
DC = 1000
NCHD = E // DC // NS
ZCH = 1000
ZR = GC
BLK = 2000

_mesh = plsc.VectorSubcoreMesh(core_axis_name="c", subcore_axis_name="s")
_sc_params = pltpu.CompilerParams(use_tc_tiling_on_sc=False)


@functools.partial(
    pl.kernel,
    out_type=jax.ShapeDtypeStruct((NC, 1, N), jnp.float32),
    mesh=_mesh,
    compiler_params=_sc_params,
    scratch_types=[
        pltpu.VMEM((NCHD, DC), jnp.int32),
        pltpu.VMEM((DC,), jnp.float32),
        pltpu.VMEM((ZCH,), jnp.float32),
        pltpu.VMEM((N,), jnp.float32),
        pltpu.VMEM_SHARED((N,), jnp.float32),
        pltpu.SemaphoreType.DMA,
    ],
)
def _deg_kernel(dst2_hbm, ones_hbm, zeros_hbm, out_hbm,
                idx_v, ones_v, zb, wb, acc, sem):
    c = lax.axis_index("c")
    s = lax.axis_index("s")

    pltpu.sync_copy(zeros_hbm, zb)
    for j in range(N // ZCH):
        @pl.when(s == (j % NS))
        def _(j=j):
            pltpu.sync_copy(zb, acc.at[pl.ds(j * ZCH, ZCH)])

    pltpu.sync_copy(ones_hbm, ones_v)
    plsc.subcore_barrier()

    pltpu.sync_copy(dst2_hbm.at[c, pl.ds(s * NCHD, NCHD)], idx_v)

    def body(j, carry):
        pltpu.sync_copy(ones_v, acc.at[idx_v.at[j]], add=True)
        return carry

    lax.fori_loop(0, NCHD, body, 0)
    plsc.subcore_barrier()

    @pl.when(s == 0)
    def _():
        pltpu.sync_copy(acc, wb)
        pltpu.sync_copy(wb, out_hbm.at[c, 0])


@functools.partial(
    pl.kernel,
    out_type=jax.ShapeDtypeStruct((NC, N, H), jnp.float32),
    mesh=_mesh,
    compiler_params=_sc_params,
    scratch_types=[
        pltpu.VMEM((NCH, GC), jnp.int32),
        pltpu.VMEM((NCH, GC), jnp.int32),
        pltpu.VMEM((GC, H), jnp.float32),
        pltpu.VMEM((GC, H), jnp.float32),
        pltpu.VMEM_SHARED((N, H), jnp.float32),
        pltpu.SemaphoreType.DMA,
        pltpu.SemaphoreType.DMA,
        pltpu.SemaphoreType.DMA,
        pltpu.SemaphoreType.DMA,
    ],
)
def _scatter_kernel(tab_hbm, ei_hbm, zrows_hbm, out_hbm,
                    idx_s, idx_d, rows0, rows1, acc,
                    gsem0, gsem1, ssem0, ssem1):
    c = lax.axis_index("c")
    s = lax.axis_index("s")
    wid = s * NC + c
    rows = (rows0, rows1)
    gsem = (gsem0, gsem1)
    ssem = (ssem0, ssem1)

    pltpu.sync_copy(zrows_hbm, rows0)
    for j in range(N // ZR):
        @pl.when(s == (j % NS))
        def _(j=j):
            pltpu.sync_copy(rows0, acc.at[pl.ds(j * ZR, ZR)])

    plsc.subcore_barrier()

    pltpu.sync_copy(ei_hbm.at[0, pl.ds(wid * NCH, NCH)], idx_s)
    pltpu.sync_copy(ei_hbm.at[1, pl.ds(wid * NCH, NCH)], idx_d)

    pltpu.async_copy(tab_hbm.at[idx_s.at[0]], rows[0], gsem[0])

    def pair(p, carry):
        for b in range(2):
            t = 2 * p + b
            o = 1 - b

            @pl.when(t > 0)
            def _(o=o):
                pltpu.make_async_copy(zrows_hbm, rows[o], ssem[o]).wait()

            @pl.when(t + 1 < NCH)
            def _(t=t, o=o):
                pltpu.async_copy(tab_hbm.at[idx_s.at[t + 1]], rows[o],
                                 gsem[o])

            pltpu.make_async_copy(zrows_hbm, rows[b], gsem[b]).wait()
            pltpu.async_copy(rows[b], acc.at[idx_d.at[t]], ssem[b],
                             add=True)
        return carry

    lax.fori_loop(0, NCH // 2, pair, 0)
    pltpu.make_async_copy(zrows_hbm, rows[1], ssem[1]).wait()

    plsc.subcore_barrier()
    for j in range(N // ZR):
        @pl.when(s == (j % NS))
        def _(j=j):
            pltpu.sync_copy(acc.at[pl.ds(j * ZR, ZR)], rows0)
            pltpu.sync_copy(rows0, out_hbm.at[c, pl.ds(j * ZR, ZR)])


def _dinv(dp):
    return lax.rsqrt(dp + 1.0)


def _tc_a_body(x_ref, w_ref, b_ref, dp_ref, h_ref, g_ref):
    h = jnp.dot(x_ref[...], w_ref[...],
                preferred_element_type=jnp.float32) + b_ref[...]
    dinv = _dinv(dp_ref[...])
    h_ref[...] = h
    g_ref[...] = dinv * h


_tc_a = pl.pallas_call(
    _tc_a_body,
    grid=(N // BLK,),
    in_specs=[
        pl.BlockSpec((BLK, D), lambda i: (i, 0)),
        pl.BlockSpec((D, H), lambda i: (0, 0)),
        pl.BlockSpec((1, H), lambda i: (0, 0)),
        pl.BlockSpec((BLK, 1), lambda i: (i, 0)),
    ],
    out_specs=[pl.BlockSpec((BLK, H), lambda i: (i, 0))] * 2,
    out_shape=[jax.ShapeDtypeStruct((N, H), jnp.float32)] * 2,
)


def _tc_b_body(sp_ref, h1_ref, dp_ref, w_ref, b_ref, h2_ref, g2_ref):
    dinv = _dinv(dp_ref[...])
    ssum = sp_ref[0] + sp_ref[1]
    z = jnp.maximum(dinv * ssum + (dinv * dinv) * h1_ref[...], 0.0)
    h2 = jnp.dot(z, w_ref[...],
                 preferred_element_type=jnp.float32) + b_ref[...]
    h2_ref[...] = h2
    g2_ref[...] = dinv * h2


_tc_b = pl.pallas_call(
    _tc_b_body,
    grid=(N // BLK,),
    in_specs=[
        pl.BlockSpec((NC, BLK, H), lambda i: (0, i, 0)),
        pl.BlockSpec((BLK, H), lambda i: (i, 0)),
        pl.BlockSpec((BLK, 1), lambda i: (i, 0)),
        pl.BlockSpec((H, H), lambda i: (0, 0)),
        pl.BlockSpec((1, H), lambda i: (0, 0)),
    ],
    out_specs=[pl.BlockSpec((BLK, H), lambda i: (i, 0))] * 2,
    out_shape=[jax.ShapeDtypeStruct((N, H), jnp.float32)] * 2,
)


def _tc_c_body(sp_ref, h2_ref, dp_ref, o_ref):
    dinv = _dinv(dp_ref[...])
    ssum = sp_ref[0] + sp_ref[1]
    o_ref[...] = dinv * ssum + (dinv * dinv) * h2_ref[...]


_tc_c = pl.pallas_call(
    _tc_c_body,
    grid=(N // BLK,),
    in_specs=[
        pl.BlockSpec((NC, BLK, H), lambda i: (0, i, 0)),
        pl.BlockSpec((BLK, H), lambda i: (i, 0)),
        pl.BlockSpec((BLK, 1), lambda i: (i, 0)),
    ],
    out_specs=pl.BlockSpec((BLK, H), lambda i: (i, 0)),
    out_shape=jax.ShapeDtypeStruct((N, H), jnp.float32),
)


def kernel(features_plus, features_minus, edge_index_pos, edge_index_neg,
           Wp1, bp1, Wp2, bp2, Wn1, bn1, Wn2, bn2):
    eip = edge_index_pos.reshape(2, NCHG, GC)
    ein = edge_index_neg.reshape(2, NCHG, GC)
    ones = jnp.ones((DC,), jnp.float32)
    zeros1 = jnp.zeros((ZCH,), jnp.float32)
    zrows = jnp.zeros((ZR, H), jnp.float32)

    dst2 = jnp.stack([edge_index_pos[1], edge_index_neg[1]]
                     ).reshape(2, E // DC, DC)
    degb = _deg_kernel(dst2, ones, zeros1)
    dpp = degb[0].reshape(N, 1)
    dpn = degb[1].reshape(N, 1)

    h1p, g1p = _tc_a(features_plus, Wp1, bp1.reshape(1, H), dpp)
    h1n, g1n = _tc_a(features_minus, Wn1, bn1.reshape(1, H), dpn)

    s1p = _scatter_kernel(g1p, eip, zrows)
    s1n = _scatter_kernel(g1n, ein, zrows)

    h2p, g2p = _tc_b(s1p, h1p, dpp, Wp2, bp2.reshape(1, H))
    h2n, g2n = _tc_b(s1n, h1n, dpn, Wn2, bn2.reshape(1, H))

    s2p = _scatter_kernel(g2p, eip, zrows)
    s2n = _scatter_kernel(g2n, ein, zrows)

    x = _tc_c(s2p, h2p, dpp)
    y = _tc_c(s2n, h2n, dpn)
    return (x, y)


# speedup vs baseline: 43.7689x; 1.1635x over previous
import functools

import jax
import jax.numpy as jnp
from jax import lax
from jax.experimental import pallas as pl
from jax.experimental.pallas import tpu as pltpu
from jax.experimental.pallas import tpu_sc as plsc

N = 10000
D = 128
H = 64
E = 320000

NC = 2
NS = 16
NW = NC * NS
GC = 125
NCHG = E // GC
NCH = NCHG // NW
DC = 1000
NCHD = E // DC // NS
ZCH = 1000
ZR = GC
BLK = 2000

_mesh = plsc.VectorSubcoreMesh(core_axis_name="c", subcore_axis_name="s")
_sc_params = pltpu.CompilerParams(use_tc_tiling_on_sc=False)


@functools.partial(
    pl.kernel,
    out_type=jax.ShapeDtypeStruct((NC, 1, N), jnp.float32),
    mesh=_mesh,
    compiler_params=_sc_params,
    scratch_types=[
        pltpu.VMEM((NCHD, DC), jnp.int32),
        pltpu.VMEM((DC,), jnp.float32),
        pltpu.VMEM((ZCH,), jnp.float32),
        pltpu.VMEM((N,), jnp.float32),
        pltpu.VMEM_SHARED((N,), jnp.float32),
        pltpu.SemaphoreType.DMA,
    ],
)
def _deg_kernel(dst2_hbm, ones_hbm, zeros_hbm, out_hbm,
                idx_v, ones_v, zb, wb, acc, sem):
    c = lax.axis_index("c")
    s = lax.axis_index("s")

    pltpu.sync_copy(zeros_hbm, zb)
    for j in range(N // ZCH):
        @pl.when(s == (j % NS))
        def _(j=j):
            pltpu.sync_copy(zb, acc.at[pl.ds(j * ZCH, ZCH)])

    pltpu.sync_copy(ones_hbm, ones_v)
    plsc.subcore_barrier()

    pltpu.sync_copy(dst2_hbm.at[c, pl.ds(s * NCHD, NCHD)], idx_v)

    def body(j, carry):
        pltpu.sync_copy(ones_v, acc.at[idx_v.at[j]], add=True)
        return carry

    lax.fori_loop(0, NCHD, body, 0)
    plsc.subcore_barrier()

    @pl.when(s == 0)
    def _():
        pltpu.sync_copy(acc, wb)
        pltpu.sync_copy(wb, out_hbm.at[c, 0])


@functools.partial(
    pl.kernel,
    out_type=jax.ShapeDtypeStruct((NC, N, H), jnp.float32),
    mesh=_mesh,
    compiler_params=_sc_params,
    scratch_types=[
        pltpu.VMEM((NCH, GC), jnp.int32),
        pltpu.VMEM((NCH, GC), jnp.int32),
        pltpu.VMEM((GC, H), jnp.float32),
        pltpu.VMEM((GC, H), jnp.float32),
        pltpu.VMEM((GC, H), jnp.float32),
        pltpu.VMEM((GC, H), jnp.float32),
        pltpu.VMEM_SHARED((N, H), jnp.float32),
        pltpu.SemaphoreType.DMA,
        pltpu.SemaphoreType.DMA,
        pltpu.SemaphoreType.DMA,
        pltpu.SemaphoreType.DMA,
        pltpu.SemaphoreType.DMA,
        pltpu.SemaphoreType.DMA,
        pltpu.SemaphoreType.DMA,
        pltpu.SemaphoreType.DMA,
    ],
)
def _scatter_kernel(tab_hbm, ei_hbm, zrows_hbm, out_hbm,
                    idx_s, idx_d, rows0, rows1, rows2, rows3, acc,
                    gsem0, gsem1, gsem2, gsem3, ssem0, ssem1, ssem2, ssem3):
    c = lax.axis_index("c")
    s = lax.axis_index("s")
    wid = s * NC + c
    rows = (rows0, rows1, rows2, rows3)
    gsem = (gsem0, gsem1, gsem2, gsem3)
    ssem = (ssem0, ssem1, ssem2, ssem3)

    pltpu.sync_copy(zrows_hbm, rows0)
    for j in range(N // ZR):
        @pl.when(s == (j % NS))
        def _(j=j):
            pltpu.sync_copy(rows0, acc.at[pl.ds(j * ZR, ZR)])

    plsc.subcore_barrier()

    pltpu.sync_copy(ei_hbm.at[0, pl.ds(wid * NCH, NCH)], idx_s)
    pltpu.sync_copy(ei_hbm.at[1, pl.ds(wid * NCH, NCH)], idx_d)

    for b in range(3):
        pltpu.async_copy(tab_hbm.at[idx_s.at[b]], rows[b], gsem[b])

    def ring(p, carry):
        for b in range(4):
            t = 4 * p + b
            a = (b + 3) % 4

            @pl.when(t > 0)
            def _(a=a):
                pltpu.make_async_copy(zrows_hbm, rows[a], ssem[a]).wait()

            @pl.when(t + 3 < NCH)
            def _(t=t, a=a):
                pltpu.async_copy(tab_hbm.at[idx_s.at[t + 3]], rows[a],
                                 gsem[a])

            pltpu.make_async_copy(zrows_hbm, rows[b], gsem[b]).wait()
            pltpu.async_copy(rows[b], acc.at[idx_d.at[t]], ssem[b],
                             add=True)
        return carry

    lax.fori_loop(0, NCH // 4, ring, 0)
    pltpu.make_async_copy(zrows_hbm, rows[(NCH - 1) % 4], ssem[(NCH - 1) % 4]).wait()

    plsc.subcore_barrier()
    for j in range(N // ZR):
        @pl.when(s == (j % NS))
        def _(j=j):
            pltpu.sync_copy(acc.at[pl.ds(j * ZR, ZR)], rows0)
            pltpu.sync_copy(rows0, out_hbm.at[c, pl.ds(j * ZR, ZR)])


def _dinv(dp):
    return lax.rsqrt(dp + 1.0)


def _tc_a_body(x_ref, w_ref, b_ref, dp_ref, h_ref, g_ref):
    h = jnp.dot(x_ref[...], w_ref[...],
                preferred_element_type=jnp.float32) + b_ref[...]
    dinv = _dinv(dp_ref[...])
    h_ref[...] = h
    g_ref[...] = dinv * h


_tc_a = pl.pallas_call(
    _tc_a_body,
    grid=(N // BLK,),
    in_specs=[
        pl.BlockSpec((BLK, D), lambda i: (i, 0)),
        pl.BlockSpec((D, H), lambda i: (0, 0)),
        pl.BlockSpec((1, H), lambda i: (0, 0)),
        pl.BlockSpec((BLK, 1), lambda i: (i, 0)),
    ],
    out_specs=[pl.BlockSpec((BLK, H), lambda i: (i, 0))] * 2,
    out_shape=[jax.ShapeDtypeStruct((N, H), jnp.float32)] * 2,
)


def _tc_b_body(sp_ref, h1_ref, dp_ref, w_ref, b_ref, h2_ref, g2_ref):
    dinv = _dinv(dp_ref[...])
    ssum = sp_ref[0] + sp_ref[1]
    z = jnp.maximum(dinv * ssum + (dinv * dinv) * h1_ref[...], 0.0)
    h2 = jnp.dot(z, w_ref[...],
                 preferred_element_type=jnp.float32) + b_ref[...]
    h2_ref[...] = h2
    g2_ref[...] = dinv * h2


_tc_b = pl.pallas_call(
    _tc_b_body,
    grid=(N // BLK,),
    in_specs=[
        pl.BlockSpec((NC, BLK, H), lambda i: (0, i, 0)),
        pl.BlockSpec((BLK, H), lambda i: (i, 0)),
        pl.BlockSpec((BLK, 1), lambda i: (i, 0)),
        pl.BlockSpec((H, H), lambda i: (0, 0)),
        pl.BlockSpec((1, H), lambda i: (0, 0)),
    ],
    out_specs=[pl.BlockSpec((BLK, H), lambda i: (i, 0))] * 2,
    out_shape=[jax.ShapeDtypeStruct((N, H), jnp.float32)] * 2,
)


def _tc_c_body(sp_ref, h2_ref, dp_ref, o_ref):
    dinv = _dinv(dp_ref[...])
    ssum = sp_ref[0] + sp_ref[1]
    o_ref[...] = dinv * ssum + (dinv * dinv) * h2_ref[...]


_tc_c = pl.pallas_call(
    _tc_c_body,
    grid=(N // BLK,),
    in_specs=[
        pl.BlockSpec((NC, BLK, H), lambda i: (0, i, 0)),
        pl.BlockSpec((BLK, H), lambda i: (i, 0)),
        pl.BlockSpec((BLK, 1), lambda i: (i, 0)),
    ],
    out_specs=pl.BlockSpec((BLK, H), lambda i: (i, 0)),
    out_shape=jax.ShapeDtypeStruct((N, H), jnp.float32),
)


def kernel(features_plus, features_minus, edge_index_pos, edge_index_neg,
           Wp1, bp1, Wp2, bp2, Wn1, bn1, Wn2, bn2):
    eip = edge_index_pos.reshape(2, NCHG, GC)
    ein = edge_index_neg.reshape(2, NCHG, GC)
    ones = jnp.ones((DC,), jnp.float32)
    zeros1 = jnp.zeros((ZCH,), jnp.float32)
    zrows = jnp.zeros((ZR, H), jnp.float32)

    dst2 = jnp.stack([edge_index_pos[1], edge_index_neg[1]]
                     ).reshape(2, E // DC, DC)
    degb = _deg_kernel(dst2, ones, zeros1)
    dpp = degb[0].reshape(N, 1)
    dpn = degb[1].reshape(N, 1)

    h1p, g1p = _tc_a(features_plus, Wp1, bp1.reshape(1, H), dpp)
    h1n, g1n = _tc_a(features_minus, Wn1, bn1.reshape(1, H), dpn)

    s1p = _scatter_kernel(g1p, eip, zrows)
    s1n = _scatter_kernel(g1n, ein, zrows)

    h2p, g2p = _tc_b(s1p, h1p, dpp, Wp2, bp2.reshape(1, H))
    h2n, g2n = _tc_b(s1n, h1n, dpn, Wn2, bn2.reshape(1, H))

    s2p = _scatter_kernel(g2p, eip, zrows)
    s2n = _scatter_kernel(g2n, ein, zrows)

    x = _tc_c(s2p, h2p, dpp)
    y = _tc_c(s2n, h2n, dpn)
    return (x, y)


# speedup vs baseline: 49.6732x; 1.1349x over previous
import functools

import jax
import jax.numpy as jnp
from jax import lax
from jax.experimental import pallas as pl
from jax.experimental.pallas import tpu as pltpu
from jax.experimental.pallas import tpu_sc as plsc

N = 10000
D = 128
H = 64
E = 320000

NC = 2
NS = 16
NW = NC * NS
GC = 125
NCHG = E // GC
NCH = NCHG // NW
DC = 1000
NCHD = E // DC // NS
ZCH = 1000
ZR = GC
BLK = 2000

_mesh = plsc.VectorSubcoreMesh(core_axis_name="c", subcore_axis_name="s")
_sc_params = pltpu.CompilerParams(use_tc_tiling_on_sc=False)


@functools.partial(
    pl.kernel,
    out_type=jax.ShapeDtypeStruct((NC, 1, N), jnp.float32),
    mesh=_mesh,
    compiler_params=_sc_params,
    scratch_types=[
        pltpu.VMEM((NCHD, DC), jnp.int32),
        pltpu.VMEM((DC,), jnp.float32),
        pltpu.VMEM((ZCH,), jnp.float32),
        pltpu.VMEM((N,), jnp.float32),
        pltpu.VMEM_SHARED((N,), jnp.float32),
        pltpu.SemaphoreType.DMA,
    ],
)
def _deg_kernel(dst2_hbm, ones_hbm, zeros_hbm, out_hbm,
                idx_v, ones_v, zb, wb, acc, sem):
    c = lax.axis_index("c")
    s = lax.axis_index("s")

    pltpu.sync_copy(zeros_hbm, zb)
    for j in range(N // ZCH):
        @pl.when(s == (j % NS))
        def _(j=j):
            pltpu.sync_copy(zb, acc.at[pl.ds(j * ZCH, ZCH)])

    pltpu.sync_copy(ones_hbm, ones_v)
    plsc.subcore_barrier()

    pltpu.sync_copy(dst2_hbm.at[c, pl.ds(s * NCHD, NCHD)], idx_v)

    def body(j, carry):
        pltpu.sync_copy(ones_v, acc.at[idx_v.at[j]], add=True)
        return carry

    lax.fori_loop(0, NCHD, body, 0)
    plsc.subcore_barrier()

    @pl.when(s == 0)
    def _():
        pltpu.sync_copy(acc, wb)
        pltpu.sync_copy(wb, out_hbm.at[c, 0])


@functools.partial(
    pl.kernel,
    out_type=jax.ShapeDtypeStruct((NC, N, H), jnp.float32),
    mesh=_mesh,
    compiler_params=_sc_params,
    scratch_types=[
        pltpu.VMEM((NCH, GC), jnp.int32),
        pltpu.VMEM((NCH, GC), jnp.int32),
        pltpu.VMEM((GC, H), jnp.float32),
        pltpu.VMEM((GC, H), jnp.float32),
        pltpu.VMEM((GC, H), jnp.float32),
        pltpu.VMEM((GC, H), jnp.float32),
        pltpu.VMEM_SHARED((N, H), jnp.float32),
        pltpu.SemaphoreType.DMA,
        pltpu.SemaphoreType.DMA,
        pltpu.SemaphoreType.DMA,
        pltpu.SemaphoreType.DMA,
        pltpu.SemaphoreType.DMA,
        pltpu.SemaphoreType.DMA,
        pltpu.SemaphoreType.DMA,
        pltpu.SemaphoreType.DMA,
    ],
)
def _scatter_kernel(tab_hbm, ei_hbm, zrows_hbm, out_hbm,
                    idx_s, idx_d, rows0, rows1, rows2, rows3, acc,
                    gsem0, gsem1, gsem2, gsem3, ssem0, ssem1, ssem2, ssem3):
    c = lax.axis_index("c")
    s = lax.axis_index("s")
    wid = s * NC + c
    rows = (rows0, rows1, rows2, rows3)
    gsem = (gsem0, gsem1, gsem2, gsem3)
    ssem = (ssem0, ssem1, ssem2, ssem3)

    pltpu.sync_copy(zrows_hbm, rows0)
    for j in range(N // ZR):
        @pl.when(s == (j % NS))
        def _(j=j):
            pltpu.sync_copy(rows0, acc.at[pl.ds(j * ZR, ZR)])

    plsc.subcore_barrier()

    pltpu.sync_copy(ei_hbm.at[0, pl.ds(wid * NCH, NCH)], idx_s)
    pltpu.sync_copy(ei_hbm.at[1, pl.ds(wid * NCH, NCH)], idx_d)

    for b in range(3):
        pltpu.async_copy(tab_hbm.at[idx_s.at[b]], rows[b], gsem[b])

    def ring(p, carry):
        for b in range(4):
            t = 4 * p + b
            a = (b + 3) % 4

            @pl.when(t > 0)
            def _(a=a):
                pltpu.make_async_copy(zrows_hbm, rows[a], ssem[a]).wait()

            @pl.when(t + 3 < NCH)
            def _(t=t, a=a):
                pltpu.async_copy(tab_hbm.at[idx_s.at[t + 3]], rows[a],
                                 gsem[a])

            pltpu.make_async_copy(zrows_hbm, rows[b], gsem[b]).wait()
            pltpu.async_copy(rows[b], acc.at[idx_d.at[t]], ssem[b],
                             add=True)
        return carry

    lax.fori_loop(0, NCH // 4, ring, 0)
    pltpu.make_async_copy(zrows_hbm, rows[(NCH - 1) % 4], ssem[(NCH - 1) % 4]).wait()

    plsc.subcore_barrier()
    for j in range(N // ZR):
        @pl.when(s == (j % NS))
        def _(j=j):
            pltpu.sync_copy(acc.at[pl.ds(j * ZR, ZR)], rows0)
            pltpu.sync_copy(rows0, out_hbm.at[c, pl.ds(j * ZR, ZR)])


N2 = N // 2
H2 = 2 * H
BLK2 = 1000


def _scale(dp):
    dinv = lax.rsqrt(dp + 1.0)
    return jnp.concatenate(
        [jnp.broadcast_to(dinv[:, 0:1], (dp.shape[0], H)),
         jnp.broadcast_to(dinv[:, 1:2], (dp.shape[0], H))], axis=1)


def _tc_a_body(x_ref, w_ref, b_ref, dp_ref, h_ref, g_ref):
    h = jnp.dot(x_ref[...], w_ref[...],
                preferred_element_type=jnp.float32) + b_ref[...]
    sc = _scale(dp_ref[...])
    h_ref[...] = h
    g_ref[...] = sc * h


_tc_a = pl.pallas_call(
    _tc_a_body,
    grid=(N2 // BLK2,),
    in_specs=[
        pl.BlockSpec((BLK2, 2 * D), lambda i: (i, 0)),
        pl.BlockSpec((2 * D, H2), lambda i: (0, 0)),
        pl.BlockSpec((1, H2), lambda i: (0, 0)),
        pl.BlockSpec((BLK2, 2), lambda i: (i, 0)),
    ],
    out_specs=[pl.BlockSpec((BLK2, H2), lambda i: (i, 0))] * 2,
    out_shape=[jax.ShapeDtypeStruct((N2, H2), jnp.float32)] * 2,
)


def _tc_b_body(sp_ref, h1_ref, dp_ref, w_ref, b_ref, h2_ref, g2_ref):
    sc = _scale(dp_ref[...])
    ssum = sp_ref[0] + sp_ref[1]
    z = jnp.maximum(sc * ssum + (sc * sc) * h1_ref[...], 0.0)
    h2 = jnp.dot(z, w_ref[...],
                 preferred_element_type=jnp.float32) + b_ref[...]
    h2_ref[...] = h2
    g2_ref[...] = sc * h2


_tc_b = pl.pallas_call(
    _tc_b_body,
    grid=(N2 // BLK2,),
    in_specs=[
        pl.BlockSpec((NC, BLK2, H2), lambda i: (0, i, 0)),
        pl.BlockSpec((BLK2, H2), lambda i: (i, 0)),
        pl.BlockSpec((BLK2, 2), lambda i: (i, 0)),
        pl.BlockSpec((H2, H2), lambda i: (0, 0)),
        pl.BlockSpec((1, H2), lambda i: (0, 0)),
    ],
    out_specs=[pl.BlockSpec((BLK2, H2), lambda i: (i, 0))] * 2,
    out_shape=[jax.ShapeDtypeStruct((N2, H2), jnp.float32)] * 2,
)


def _tc_c_body(sp_ref, h2_ref, dp_ref, o_ref):
    sc = _scale(dp_ref[...])
    ssum = sp_ref[0] + sp_ref[1]
    o_ref[...] = sc * ssum + (sc * sc) * h2_ref[...]


_tc_c = pl.pallas_call(
    _tc_c_body,
    grid=(N2 // BLK2,),
    in_specs=[
        pl.BlockSpec((NC, BLK2, H2), lambda i: (0, i, 0)),
        pl.BlockSpec((BLK2, H2), lambda i: (i, 0)),
        pl.BlockSpec((BLK2, 2), lambda i: (i, 0)),
    ],
    out_specs=pl.BlockSpec((BLK2, H2), lambda i: (i, 0)),
    out_shape=jax.ShapeDtypeStruct((N2, H2), jnp.float32),
)


def _bd(W):
    z = jnp.zeros_like(W)
    return jnp.concatenate(
        [jnp.concatenate([W, z], axis=1), jnp.concatenate([z, W], axis=1)],
        axis=0)


def _bt(b):
    return jnp.concatenate([b, b]).reshape(1, H2)


def kernel(features_plus, features_minus, edge_index_pos, edge_index_neg,
           Wp1, bp1, Wp2, bp2, Wn1, bn1, Wn2, bn2):
    eip = edge_index_pos.reshape(2, NCHG, GC)
    ein = edge_index_neg.reshape(2, NCHG, GC)
    ones = jnp.ones((DC,), jnp.float32)
    zeros1 = jnp.zeros((ZCH,), jnp.float32)
    zrows = jnp.zeros((ZR, H), jnp.float32)

    dst2 = jnp.stack([edge_index_pos[1], edge_index_neg[1]]
                     ).reshape(2, E // DC, DC)
    degb = _deg_kernel(dst2, ones, zeros1)
    dpp = degb[0].reshape(N2, 2)
    dpn = degb[1].reshape(N2, 2)

    xp2 = features_plus.reshape(N2, 2 * D)
    xn2 = features_minus.reshape(N2, 2 * D)

    h1p, g1p = _tc_a(xp2, _bd(Wp1), _bt(bp1), dpp)
    h1n, g1n = _tc_a(xn2, _bd(Wn1), _bt(bn1), dpn)

    s1p = _scatter_kernel(g1p.reshape(N, H), eip, zrows)
    s1n = _scatter_kernel(g1n.reshape(N, H), ein, zrows)

    h2p, g2p = _tc_b(s1p.reshape(NC, N2, H2), h1p, dpp, _bd(Wp2), _bt(bp2))
    h2n, g2n = _tc_b(s1n.reshape(NC, N2, H2), h1n, dpn, _bd(Wn2), _bt(bn2))

    s2p = _scatter_kernel(g2p.reshape(N, H), eip, zrows)
    s2n = _scatter_kernel(g2n.reshape(N, H), ein, zrows)

    x = _tc_c(s2p.reshape(NC, N2, H2), h2p, dpp)
    y = _tc_c(s2n.reshape(NC, N2, H2), h2n, dpn)
    return (x.reshape(N, H), y.reshape(N, H))


# speedup vs baseline: 50.5683x; 1.0180x over previous
import functools

import jax
import jax.numpy as jnp
from jax import lax
from jax.experimental import pallas as pl
from jax.experimental.pallas import tpu as pltpu
from jax.experimental.pallas import tpu_sc as plsc

N = 10000
D = 128
H = 64
E = 320000

NC = 2
NS = 16
NW = NC * NS
GC = 125
NCHG = E // GC
NCH = NCHG // NW
DC = 1000
NCHD = E // DC // NS
ZCH = 1000
ZR = GC
BLK = 2000

_mesh = plsc.VectorSubcoreMesh(core_axis_name="c", subcore_axis_name="s")
_sc_params = pltpu.CompilerParams(use_tc_tiling_on_sc=False)


@functools.partial(
    pl.kernel,
    out_type=jax.ShapeDtypeStruct((NC, 1, N), jnp.float32),
    mesh=_mesh,
    compiler_params=_sc_params,
    scratch_types=[
        pltpu.VMEM((NCHD, DC), jnp.int32),
        pltpu.VMEM((DC,), jnp.float32),
        pltpu.VMEM((ZCH,), jnp.float32),
        pltpu.VMEM((N,), jnp.float32),
        pltpu.VMEM_SHARED((N,), jnp.float32),
        pltpu.SemaphoreType.DMA,
    ],
)
def _deg_kernel(dst2_hbm, ones_hbm, zeros_hbm, out_hbm,
                idx_v, ones_v, zb, wb, acc, sem):
    c = lax.axis_index("c")
    s = lax.axis_index("s")

    pltpu.sync_copy(zeros_hbm, zb)
    for j in range(N // ZCH):
        @pl.when(s == (j % NS))
        def _(j=j):
            pltpu.sync_copy(zb, acc.at[pl.ds(j * ZCH, ZCH)])

    pltpu.sync_copy(ones_hbm, ones_v)
    plsc.subcore_barrier()

    pltpu.sync_copy(dst2_hbm.at[c, pl.ds(s * NCHD, NCHD)], idx_v)

    def body(j, carry):
        pltpu.sync_copy(ones_v, acc.at[idx_v.at[j]], add=True)
        return carry

    lax.fori_loop(0, NCHD, body, 0)
    plsc.subcore_barrier()

    @pl.when(s == 0)
    def _():
        pltpu.sync_copy(acc, wb)
        pltpu.sync_copy(wb, out_hbm.at[c, 0])


@functools.partial(
    pl.kernel,
    out_type=jax.ShapeDtypeStruct((NC, N, H), jnp.float32),
    mesh=_mesh,
    compiler_params=_sc_params,
    scratch_types=[
        pltpu.VMEM((NCH, GC), jnp.int32),
        pltpu.VMEM((NCH, GC), jnp.int32),
        pltpu.VMEM((GC, H), jnp.float32),
        pltpu.VMEM((GC, H), jnp.float32),
        pltpu.VMEM((GC, H), jnp.float32),
        pltpu.VMEM((GC, H), jnp.float32),
        pltpu.VMEM_SHARED((N, H), jnp.float32),
        pltpu.SemaphoreType.DMA,
        pltpu.SemaphoreType.DMA,
        pltpu.SemaphoreType.DMA,
        pltpu.SemaphoreType.DMA,
        pltpu.SemaphoreType.DMA,
        pltpu.SemaphoreType.DMA,
        pltpu.SemaphoreType.DMA,
        pltpu.SemaphoreType.DMA,
    ],
)
def _scatter_kernel(tab_hbm, ei_hbm, zrows_hbm, out_hbm,
                    idx_s, idx_d, rows0, rows1, rows2, rows3, acc,
                    gsem0, gsem1, gsem2, gsem3, ssem0, ssem1, ssem2, ssem3):
    c = lax.axis_index("c")
    s = lax.axis_index("s")
    wid = s * NC + c
    rows = (rows0, rows1, rows2, rows3)
    gsem = (gsem0, gsem1, gsem2, gsem3)
    ssem = (ssem0, ssem1, ssem2, ssem3)

    pltpu.sync_copy(zrows_hbm, rows0)
    for j in range(N // ZR):
        @pl.when(s == (j % NS))
        def _(j=j):
            pltpu.sync_copy(rows0, acc.at[pl.ds(j * ZR, ZR)])

    plsc.subcore_barrier()

    pltpu.sync_copy(ei_hbm.at[0, pl.ds(wid * NCH, NCH)], idx_s)
    pltpu.sync_copy(ei_hbm.at[1, pl.ds(wid * NCH, NCH)], idx_d)

    for b in range(3):
        pltpu.async_copy(tab_hbm.at[idx_s.at[b]], rows[b], gsem[b])

    def ring(p, carry):
        for b in range(4):
            t = 4 * p + b
            a = (b + 3) % 4

            @pl.when(t > 0)
            def _(a=a):
                pltpu.make_async_copy(zrows_hbm, rows[a], ssem[a]).wait()

            @pl.when(t + 3 < NCH)
            def _(t=t, a=a):
                pltpu.async_copy(tab_hbm.at[idx_s.at[t + 3]], rows[a],
                                 gsem[a])

            pltpu.make_async_copy(zrows_hbm, rows[b], gsem[b]).wait()
            pltpu.async_copy(rows[b], acc.at[idx_d.at[t]], ssem[b],
                             add=True)
        return carry

    lax.fori_loop(0, NCH // 4, ring, 0)
    pltpu.make_async_copy(zrows_hbm, rows[(NCH - 1) % 4], ssem[(NCH - 1) % 4]).wait()

    plsc.subcore_barrier()
    for j in range(N // ZR):
        @pl.when(s == (j % NS))
        def _(j=j):
            pltpu.sync_copy(acc.at[pl.ds(j * ZR, ZR)], rows0)
            pltpu.sync_copy(rows0, out_hbm.at[c, pl.ds(j * ZR, ZR)])


N2 = N // 2
H2 = 2 * H
BLK2 = 1000


def _scale(dp):
    dinv = lax.rsqrt(dp + 1.0)
    return jnp.concatenate(
        [jnp.broadcast_to(dinv[:, 0:1], (dp.shape[0], H)),
         jnp.broadcast_to(dinv[:, 1:2], (dp.shape[0], H))], axis=1)


def _pair_dot(x, w, b):
    he = jnp.dot(x[:, :x.shape[1] // 2], w,
                 preferred_element_type=jnp.float32) + b
    ho = jnp.dot(x[:, x.shape[1] // 2:], w,
                 preferred_element_type=jnp.float32) + b
    return jnp.concatenate([he, ho], axis=1)


def _tc_a_body(x_ref, w_ref, b_ref, dp_ref, h_ref, g_ref):
    h = _pair_dot(x_ref[...], w_ref[...], b_ref[...])
    sc = _scale(dp_ref[...])
    h_ref[...] = h
    g_ref[...] = sc * h


_tc_a = pl.pallas_call(
    _tc_a_body,
    grid=(N2 // BLK2,),
    in_specs=[
        pl.BlockSpec((BLK2, 2 * D), lambda i: (i, 0)),
        pl.BlockSpec((D, H), lambda i: (0, 0)),
        pl.BlockSpec((1, H), lambda i: (0, 0)),
        pl.BlockSpec((BLK2, 2), lambda i: (i, 0)),
    ],
    out_specs=[pl.BlockSpec((BLK2, H2), lambda i: (i, 0))] * 2,
    out_shape=[jax.ShapeDtypeStruct((N2, H2), jnp.float32)] * 2,
)


def _tc_b_body(sp_ref, h1_ref, dp_ref, w_ref, b_ref, h2_ref, g2_ref):
    sc = _scale(dp_ref[...])
    ssum = sp_ref[0] + sp_ref[1]
    z = jnp.maximum(sc * ssum + (sc * sc) * h1_ref[...], 0.0)
    h2 = _pair_dot(z, w_ref[...], b_ref[...])
    h2_ref[...] = h2
    g2_ref[...] = sc * h2


_tc_b = pl.pallas_call(
    _tc_b_body,
    grid=(N2 // BLK2,),
    in_specs=[
        pl.BlockSpec((NC, BLK2, H2), lambda i: (0, i, 0)),
        pl.BlockSpec((BLK2, H2), lambda i: (i, 0)),
        pl.BlockSpec((BLK2, 2), lambda i: (i, 0)),
        pl.BlockSpec((H, H), lambda i: (0, 0)),
        pl.BlockSpec((1, H), lambda i: (0, 0)),
    ],
    out_specs=[pl.BlockSpec((BLK2, H2), lambda i: (i, 0))] * 2,
    out_shape=[jax.ShapeDtypeStruct((N2, H2), jnp.float32)] * 2,
)


def _tc_c_body(sp_ref, h2_ref, dp_ref, o_ref):
    sc = _scale(dp_ref[...])
    ssum = sp_ref[0] + sp_ref[1]
    o_ref[...] = sc * ssum + (sc * sc) * h2_ref[...]


_tc_c = pl.pallas_call(
    _tc_c_body,
    grid=(N2 // BLK2,),
    in_specs=[
        pl.BlockSpec((NC, BLK2, H2), lambda i: (0, i, 0)),
        pl.BlockSpec((BLK2, H2), lambda i: (i, 0)),
        pl.BlockSpec((BLK2, 2), lambda i: (i, 0)),
    ],
    out_specs=pl.BlockSpec((BLK2, H2), lambda i: (i, 0)),
    out_shape=jax.ShapeDtypeStruct((N2, H2), jnp.float32),
)


def kernel(features_plus, features_minus, edge_index_pos, edge_index_neg,
           Wp1, bp1, Wp2, bp2, Wn1, bn1, Wn2, bn2):
    eip = edge_index_pos.reshape(2, NCHG, GC)
    ein = edge_index_neg.reshape(2, NCHG, GC)
    ones = jnp.ones((DC,), jnp.float32)
    zeros1 = jnp.zeros((ZCH,), jnp.float32)
    zrows = jnp.zeros((ZR, H), jnp.float32)

    dst2 = jnp.stack([edge_index_pos[1], edge_index_neg[1]]
                     ).reshape(2, E // DC, DC)
    degb = _deg_kernel(dst2, ones, zeros1)
    dpp = degb[0].reshape(N2, 2)
    dpn = degb[1].reshape(N2, 2)

    xp2 = features_plus.reshape(N2, 2 * D)
    xn2 = features_minus.reshape(N2, 2 * D)

    h1p, g1p = _tc_a(xp2, Wp1, bp1.reshape(1, H), dpp)
    h1n, g1n = _tc_a(xn2, Wn1, bn1.reshape(1, H), dpn)

    s1p = _scatter_kernel(g1p.reshape(N, H), eip, zrows)
    s1n = _scatter_kernel(g1n.reshape(N, H), ein, zrows)

    h2p, g2p = _tc_b(s1p.reshape(NC, N2, H2), h1p, dpp, Wp2, bp2.reshape(1, H))
    h2n, g2n = _tc_b(s1n.reshape(NC, N2, H2), h1n, dpn, Wn2, bn2.reshape(1, H))

    s2p = _scatter_kernel(g2p.reshape(N, H), eip, zrows)
    s2n = _scatter_kernel(g2n.reshape(N, H), ein, zrows)

    x = _tc_c(s2p.reshape(NC, N2, H2), h2p, dpp)
    y = _tc_c(s2n.reshape(NC, N2, H2), h2n, dpn)
    return (x.reshape(N, H), y.reshape(N, H))


# speedup vs baseline: 53.5268x; 1.0585x over previous
import functools

import jax
import jax.numpy as jnp
from jax import lax
from jax.experimental import pallas as pl
from jax.experimental.pallas import tpu as pltpu
from jax.experimental.pallas import tpu_sc as plsc

N = 10000
D = 128
H = 64
E = 320000

NC = 2
NS = 16
NW = NC * NS
GC = 125
NCHG = E // GC
NCH = NCHG // NW
DC = 1000
NCHD = E // DC // NS
ZCH = 1000
ZR = GC
BLK = 2000

_mesh = plsc.VectorSubcoreMesh(core_axis_name="c", subcore_axis_name="s")
_sc_params = pltpu.CompilerParams(use_tc_tiling_on_sc=False)


@functools.partial(
    pl.kernel,
    out_type=jax.ShapeDtypeStruct((NC, 1, N), jnp.float32),
    mesh=_mesh,
    compiler_params=_sc_params,
    scratch_types=[
        pltpu.VMEM((NCHD, DC), jnp.int32),
        pltpu.VMEM((DC,), jnp.float32),
        pltpu.VMEM((ZCH,), jnp.float32),
        pltpu.VMEM((N,), jnp.float32),
        pltpu.VMEM_SHARED((N,), jnp.float32),
        pltpu.SemaphoreType.DMA,
    ],
)
def _deg_kernel(dst2_hbm, ones_hbm, zeros_hbm, out_hbm,
                idx_v, ones_v, zb, wb, acc, sem):
    c = lax.axis_index("c")
    s = lax.axis_index("s")

    pltpu.sync_copy(zeros_hbm, zb)
    for j in range(N // ZCH):
        @pl.when(s == (j % NS))
        def _(j=j):
            pltpu.sync_copy(zb, acc.at[pl.ds(j * ZCH, ZCH)])

    pltpu.sync_copy(ones_hbm, ones_v)
    plsc.subcore_barrier()

    pltpu.sync_copy(dst2_hbm.at[c, pl.ds(s * NCHD, NCHD)], idx_v)

    def body(j, carry):
        pltpu.sync_copy(ones_v, acc.at[idx_v.at[j]], add=True)
        return carry

    lax.fori_loop(0, NCHD, body, 0)
    plsc.subcore_barrier()

    @pl.when(s == 0)
    def _():
        pltpu.sync_copy(acc, wb)
        pltpu.sync_copy(wb, out_hbm.at[c, 0])


@functools.partial(
    pl.kernel,
    out_type=jax.ShapeDtypeStruct((NC, N, H), jnp.float32),
    mesh=_mesh,
    compiler_params=_sc_params,
    scratch_types=[
        pltpu.VMEM((NCH, GC), jnp.int32),
        pltpu.VMEM((NCH, GC), jnp.int32),
        pltpu.VMEM((GC, H), jnp.float32),
        pltpu.VMEM((GC, H), jnp.float32),
        pltpu.VMEM((GC, H), jnp.float32),
        pltpu.VMEM((GC, H), jnp.float32),
        pltpu.VMEM_SHARED((N, H), jnp.float32),
        pltpu.SemaphoreType.DMA,
        pltpu.SemaphoreType.DMA,
        pltpu.SemaphoreType.DMA,
        pltpu.SemaphoreType.DMA,
        pltpu.SemaphoreType.DMA,
        pltpu.SemaphoreType.DMA,
        pltpu.SemaphoreType.DMA,
        pltpu.SemaphoreType.DMA,
    ],
)
def _scatter_kernel(tab_hbm, ei_hbm, zrows_hbm, out_hbm,
                    idx_s, idx_d, rows0, rows1, rows2, rows3, acc,
                    gsem0, gsem1, gsem2, gsem3, ssem0, ssem1, ssem2, ssem3):
    c = lax.axis_index("c")
    s = lax.axis_index("s")
    wid = s * NC + c
    rows = (rows0, rows1, rows2, rows3)
    gsem = (gsem0, gsem1, gsem2, gsem3)
    ssem = (ssem0, ssem1, ssem2, ssem3)

    pltpu.async_copy(ei_hbm.at[0, pl.ds(wid * NCH, NCH)], idx_s, gsem0)
    pltpu.async_copy(ei_hbm.at[1, pl.ds(wid * NCH, NCH)], idx_d, gsem1)

    NZ = N // ZR // NS
    pltpu.sync_copy(zrows_hbm, rows0)
    for k in range(NZ):
        pltpu.async_copy(rows0, acc.at[pl.ds((NZ * s + k) * ZR, ZR)],
                         ssem[k % 4])
    for k in range(min(NZ, 4)):
        pltpu.make_async_copy(zrows_hbm, rows0, ssem[k]).wait()
    if NZ > 4:
        for k in range(4, NZ):
            pltpu.make_async_copy(zrows_hbm, rows0, ssem[k % 4]).wait()
    pltpu.make_async_copy(ei_hbm.at[0, pl.ds(0, NCH)], idx_s, gsem0).wait()
    pltpu.make_async_copy(ei_hbm.at[0, pl.ds(0, NCH)], idx_d, gsem1).wait()

    plsc.subcore_barrier()

    for b in range(3):
        pltpu.async_copy(tab_hbm.at[idx_s.at[b]], rows[b], gsem[b])

    def ring(p, carry):
        for b in range(4):
            t = 4 * p + b
            a = (b + 3) % 4

            @pl.when(t > 0)
            def _(a=a):
                pltpu.make_async_copy(zrows_hbm, rows[a], ssem[a]).wait()

            @pl.when(t + 3 < NCH)
            def _(t=t, a=a):
                pltpu.async_copy(tab_hbm.at[idx_s.at[t + 3]], rows[a],
                                 gsem[a])

            pltpu.make_async_copy(zrows_hbm, rows[b], gsem[b]).wait()
            pltpu.async_copy(rows[b], acc.at[idx_d.at[t]], ssem[b],
                             add=True)
        return carry

    lax.fori_loop(0, NCH // 4, ring, 0)
    pltpu.make_async_copy(zrows_hbm, rows[(NCH - 1) % 4], ssem[(NCH - 1) % 4]).wait()

    plsc.subcore_barrier()
    NZ2 = N // ZR // NS
    for k in range(NZ2):
        b = k % 4
        if k >= 4:
            pltpu.make_async_copy(zrows_hbm, rows[b], ssem[b]).wait()
        j = NZ2 * s + k
        pltpu.async_copy(acc.at[pl.ds(j * ZR, ZR)], rows[b], gsem[b])
        pltpu.make_async_copy(zrows_hbm, rows[b], gsem[b]).wait()
        pltpu.async_copy(rows[b], out_hbm.at[c, pl.ds(j * ZR, ZR)], ssem[b])
    for k in range(max(NZ2 - 4, 0), NZ2):
        pltpu.make_async_copy(zrows_hbm, rows[k % 4], ssem[k % 4]).wait()


N2 = N // 2
H2 = 2 * H
BLK2 = 1000


def _scale(dp):
    dinv = lax.rsqrt(dp + 1.0)
    return jnp.concatenate(
        [jnp.broadcast_to(dinv[:, 0:1], (dp.shape[0], H)),
         jnp.broadcast_to(dinv[:, 1:2], (dp.shape[0], H))], axis=1)


def _pair_dot(x, w, b):
    he = jnp.dot(x[:, :x.shape[1] // 2], w,
                 preferred_element_type=jnp.float32) + b
    ho = jnp.dot(x[:, x.shape[1] // 2:], w,
                 preferred_element_type=jnp.float32) + b
    return jnp.concatenate([he, ho], axis=1)


def _tc_a_body(x_ref, w_ref, b_ref, dp_ref, h_ref, g_ref):
    h = _pair_dot(x_ref[...], w_ref[...], b_ref[...])
    sc = _scale(dp_ref[...])
    h_ref[...] = h
    g_ref[...] = sc * h


_tc_a = pl.pallas_call(
    _tc_a_body,
    grid=(N2 // BLK2,),
    in_specs=[
        pl.BlockSpec((BLK2, 2 * D), lambda i: (i, 0)),
        pl.BlockSpec((D, H), lambda i: (0, 0)),
        pl.BlockSpec((1, H), lambda i: (0, 0)),
        pl.BlockSpec((BLK2, 2), lambda i: (i, 0)),
    ],
    out_specs=[pl.BlockSpec((BLK2, H2), lambda i: (i, 0))] * 2,
    out_shape=[jax.ShapeDtypeStruct((N2, H2), jnp.float32)] * 2,
)


def _tc_b_body(sp_ref, h1_ref, dp_ref, w_ref, b_ref, h2_ref, g2_ref):
    sc = _scale(dp_ref[...])
    ssum = sp_ref[0] + sp_ref[1]
    z = jnp.maximum(sc * ssum + (sc * sc) * h1_ref[...], 0.0)
    h2 = _pair_dot(z, w_ref[...], b_ref[...])
    h2_ref[...] = h2
    g2_ref[...] = sc * h2


_tc_b = pl.pallas_call(
    _tc_b_body,
    grid=(N2 // BLK2,),
    in_specs=[
        pl.BlockSpec((NC, BLK2, H2), lambda i: (0, i, 0)),
        pl.BlockSpec((BLK2, H2), lambda i: (i, 0)),
        pl.BlockSpec((BLK2, 2), lambda i: (i, 0)),
        pl.BlockSpec((H, H), lambda i: (0, 0)),
        pl.BlockSpec((1, H), lambda i: (0, 0)),
    ],
    out_specs=[pl.BlockSpec((BLK2, H2), lambda i: (i, 0))] * 2,
    out_shape=[jax.ShapeDtypeStruct((N2, H2), jnp.float32)] * 2,
)


def _tc_c_body(sp_ref, h2_ref, dp_ref, o_ref):
    sc = _scale(dp_ref[...])
    ssum = sp_ref[0] + sp_ref[1]
    o_ref[...] = sc * ssum + (sc * sc) * h2_ref[...]


_tc_c = pl.pallas_call(
    _tc_c_body,
    grid=(N2 // BLK2,),
    in_specs=[
        pl.BlockSpec((NC, BLK2, H2), lambda i: (0, i, 0)),
        pl.BlockSpec((BLK2, H2), lambda i: (i, 0)),
        pl.BlockSpec((BLK2, 2), lambda i: (i, 0)),
    ],
    out_specs=pl.BlockSpec((BLK2, H2), lambda i: (i, 0)),
    out_shape=jax.ShapeDtypeStruct((N2, H2), jnp.float32),
)


def kernel(features_plus, features_minus, edge_index_pos, edge_index_neg,
           Wp1, bp1, Wp2, bp2, Wn1, bn1, Wn2, bn2):
    eip = edge_index_pos.reshape(2, NCHG, GC)
    ein = edge_index_neg.reshape(2, NCHG, GC)
    ones = jnp.ones((DC,), jnp.float32)
    zeros1 = jnp.zeros((ZCH,), jnp.float32)
    zrows = jnp.zeros((ZR, H), jnp.float32)

    dst2 = jnp.stack([edge_index_pos[1], edge_index_neg[1]]
                     ).reshape(2, E // DC, DC)
    degb = _deg_kernel(dst2, ones, zeros1)
    dpp = degb[0].reshape(N2, 2)
    dpn = degb[1].reshape(N2, 2)

    xp2 = features_plus.reshape(N2, 2 * D)
    xn2 = features_minus.reshape(N2, 2 * D)

    h1p, g1p = _tc_a(xp2, Wp1, bp1.reshape(1, H), dpp)
    h1n, g1n = _tc_a(xn2, Wn1, bn1.reshape(1, H), dpn)

    s1p = _scatter_kernel(g1p.reshape(N, H), eip, zrows)
    s1n = _scatter_kernel(g1n.reshape(N, H), ein, zrows)

    h2p, g2p = _tc_b(s1p.reshape(NC, N2, H2), h1p, dpp, Wp2, bp2.reshape(1, H))
    h2n, g2n = _tc_b(s1n.reshape(NC, N2, H2), h1n, dpn, Wn2, bn2.reshape(1, H))

    s2p = _scatter_kernel(g2p.reshape(N, H), eip, zrows)
    s2n = _scatter_kernel(g2n.reshape(N, H), ein, zrows)

    x = _tc_c(s2p.reshape(NC, N2, H2), h2p, dpp)
    y = _tc_c(s2n.reshape(NC, N2, H2), h2n, dpn)
    return (x.reshape(N, H), y.reshape(N, H))
